# baseline (speedup 1.0000x reference)
import functools
import jax
import jax.numpy as jnp
from jax import lax
from jax.experimental import pallas as pl
from jax.experimental.pallas import tpu as pltpu
from jax.experimental.pallas import tpu_sc as plsc

_N_E = 1024
_E_DIM = 256
_P = 1024


def _tc_idx_body(z_ref, cb_ref, idx_ref):
    z_b = z_ref[0]
    cb = cb_ref[...]
    znorm = jnp.sum(z_b * z_b, axis=0, keepdims=True)
    enorm = jnp.sum(cb * cb, axis=1, keepdims=True)
    mm = jax.lax.dot_general(
        cb, z_b, (((1,), (0,)), ((), ())),
        preferred_element_type=jnp.float32)
    d = (znorm + enorm) - 2.0 * mm
    iota_k = jax.lax.broadcasted_iota(jnp.int32, (_N_E, _P), 0)
    dmin = jnp.min(d, axis=0, keepdims=True)
    idx_ref[0] = jnp.min(jnp.where(d == dmin, iota_k, _N_E), axis=0,
                         keepdims=True)


def _tc_transpose_body(r_ref, out_ref):
    out_ref[0] = r_ref[0].T


_NUM_SC = 2
_NUM_SUBCORES = 16


def _make_sc_gather(n_rows, d):
    nw = _NUM_SC * _NUM_SUBCORES
    ch = 128
    n_ch = n_rows // (nw * ch)
    mesh = plsc.VectorSubcoreMesh(core_axis_name="c", subcore_axis_name="s",
                                  num_cores=_NUM_SC,
                                  num_subcores=_NUM_SUBCORES)

    @functools.partial(
        pl.kernel, mesh=mesh,
        out_type=jax.ShapeDtypeStruct((n_rows, d), jnp.float32),
        scratch_types=[
            [pltpu.VMEM((ch,), jnp.int32) for _ in range(n_ch)],
            pltpu.VMEM((ch, d), jnp.float32),
            pltpu.VMEM((ch, d), jnp.float32),
            pltpu.SemaphoreType.DMA,
            pltpu.SemaphoreType.DMA,
        ],
    )
    def k(cb_hbm, idx_hbm, out_hbm, idx_vs, rows0, rows1, sem0, sem1):
        wid = lax.axis_index("s") * _NUM_SC + lax.axis_index("c")
        base = wid * n_ch
        for j in range(n_ch):
            pltpu.sync_copy(idx_hbm.at[base + j], idx_vs[j])
        rows = (rows0, rows1)
        sems = (sem0, sem1)
        cps = [pltpu.async_copy(cb_hbm.at[idx_vs[j]], rows[j], sems[j])
               for j in range(2)]
        for j in range(n_ch):
            cps[j].wait()
            pltpu.sync_copy(rows[j % 2],
                            out_hbm.at[pl.ds((base + j) * ch, ch)])
            if j + 2 < n_ch:
                cps.append(pltpu.async_copy(cb_hbm.at[idx_vs[j + 2]],
                                            rows[j % 2], sems[j % 2]))

    return k


def kernel(z, codebook):
    B, C, H, W = z.shape
    P = H * W
    z3 = z.reshape(B, C, P)
    idx = pl.pallas_call(
        _tc_idx_body,
        grid=(B,),
        in_specs=[
            pl.BlockSpec((1, C, P), lambda b: (b, 0, 0)),
            pl.BlockSpec((_N_E, _E_DIM), lambda b: (0, 0)),
        ],
        out_specs=pl.BlockSpec((1, 1, P), lambda b: (b, 0, 0)),
        out_shape=jax.ShapeDtypeStruct((B, 1, P), jnp.int32),
        compiler_params=pltpu.CompilerParams(
            dimension_semantics=("arbitrary",),
        ),
    )(z3, codebook)
    idx2 = idx.reshape(B * P // 128, 128)
    rows = _make_sc_gather(B * P, C)(codebook, idx2)
    out = pl.pallas_call(
        _tc_transpose_body,
        grid=(B,),
        in_specs=[pl.BlockSpec((1, P, C), lambda b: (b, 0, 0))],
        out_specs=pl.BlockSpec((1, C, P), lambda b: (b, 0, 0)),
        out_shape=jax.ShapeDtypeStruct((B, C, P), jnp.float32),
        compiler_params=pltpu.CompilerParams(
            dimension_semantics=("arbitrary",),
        ),
    )(rows.reshape(B, P, C))
    return out.reshape(B, C, H, W)
